# Initial kernel scaffold; baseline (speedup 1.0000x reference)
import jax
import jax.numpy as jnp
from jax.experimental import pallas as pl


def kernel(hidden_states, base_weight, gate_weight, mask_logits, atoms, expert_atom_weights, importance):
    raise NotImplementedError("write your pallas kernel here")


# speedup vs baseline: 1.0511x; 1.0511x over previous
import functools

import jax
import jax.numpy as jnp
from jax import lax
from jax.experimental import pallas as pl
from jax.experimental.pallas import tpu as pltpu

H = 1024
E = 8
K = 2
A = 64
NS = 16384
NP = H * H
T = 2048


def _router_body(x_ref, gw_ref, eaw_ref, atoms_ref, imp_ref,
                 logits_ref, rw_ref, tidx_ref, p_ref, aux_ref, w_ref):
    x = x_ref[...]
    gw = gw_ref[...]
    logits = lax.dot_general(x, gw, (((1,), (1,)), ((), ())),
                             preferred_element_type=jnp.float32)
    logits = jnp.clip(logits, -50.0, 50.0)
    logits_ref[...] = logits

    iota_e = lax.broadcasted_iota(jnp.int32, (T, E), 1)
    cnt = jnp.zeros((T, E), jnp.int32)
    for ep in range(E):
        col = logits[:, ep:ep + 1]
        gt = col > logits
        eq = (col == logits) & (ep < iota_e)
        cnt = cnt + (gt | eq).astype(jnp.int32)
    is0 = cnt == 0
    is1 = cnt == 1
    i0 = jnp.sum(jnp.where(is0, iota_e, 0), axis=1, keepdims=True)
    i1 = jnp.sum(jnp.where(is1, iota_e, 0), axis=1, keepdims=True)
    w0 = jnp.sum(jnp.where(is0, logits, 0.0), axis=1, keepdims=True)
    w1 = jnp.sum(jnp.where(is1, logits, 0.0), axis=1, keepdims=True)
    m = jnp.maximum(w0, w1)
    e0 = jnp.exp(w0 - m)
    e1 = jnp.exp(w1 - m)
    rw0 = e0 / (e0 + e1)
    rw1 = e1 / (e0 + e1)
    rw_ref[...] = jnp.concatenate([rw0, rw1], axis=1)
    tidx_ref[...] = jnp.concatenate([i0, i1], axis=1)
    p_ref[...] = jnp.where(is0, rw0, jnp.where(is1, rw1, 0.0))

    lmax = jnp.max(logits, axis=1, keepdims=True)
    ex = jnp.exp(logits - lmax)
    probs = ex / jnp.sum(ex, axis=1, keepdims=True)
    prob_mean = jnp.mean(probs, axis=0, keepdims=True)
    frac = jnp.mean((is0 | is1).astype(jnp.float32), axis=0, keepdims=True)
    aux_ref[...] = float(E) * jnp.sum(frac * prob_mean, axis=1, keepdims=True)

    eaw = eaw_ref[...]
    emax = jnp.max(eaw, axis=1, keepdims=True)
    eex = jnp.exp(eaw - emax)
    aw = eex / jnp.sum(eex, axis=1, keepdims=True)
    ad = lax.dot_general(aw, atoms_ref[...], (((1,), (0,)), ((), ())),
                         preferred_element_type=jnp.float32)
    w_ref[...] = imp_ref[...] * ad


def _router(x, gw, eaw, atoms, imp):
    return pl.pallas_call(
        _router_body,
        out_shape=(
            jax.ShapeDtypeStruct((T, E), jnp.float32),
            jax.ShapeDtypeStruct((T, K), jnp.float32),
            jax.ShapeDtypeStruct((T, K), jnp.int32),
            jax.ShapeDtypeStruct((T, E), jnp.float32),
            jax.ShapeDtypeStruct((1, 1), jnp.float32),
            jax.ShapeDtypeStruct((E, NS), jnp.float32),
        ),
    )(x, gw, eaw, atoms, imp)


def _matmul_body(x_ref, w0_ref, d_ref, p_ref, out_ref, acc_ref):
    e = pl.program_id(0)
    t = pl.program_id(1)
    xb = x_ref[pl.ds(t * 256, 256), :].astype(jnp.bfloat16)

    @pl.when(e == 0)
    def _():
        w = w0_ref[...].astype(jnp.bfloat16)
        acc_ref[pl.ds(t * 256, 256), :] = lax.dot_general(
            xb, w, (((1,), (1,)), ((), ())), preferred_element_type=jnp.float32)

    @pl.when(e > 0)
    def _():
        w = d_ref[0].astype(jnp.bfloat16)
        y = lax.dot_general(xb, w, (((1,), (1,)), ((), ())),
                            preferred_element_type=jnp.float32)
        coef = p_ref[0]
        acc_ref[pl.ds(t * 256, 256), :] += coef * y

    @pl.when(e == E)
    def _():
        out_ref[...] = acc_ref[pl.ds(t * 256, 256), :]


def _matmul(x, w0, deltas, p):
    grid = (E + 1, T // 256)
    return pl.pallas_call(
        _matmul_body,
        grid=grid,
        in_specs=[
            pl.BlockSpec((T, H), lambda e, t: (0, 0)),
            pl.BlockSpec((H, H), lambda e, t: (0, 0)),
            pl.BlockSpec((1, H, H), lambda e, t: (jnp.maximum(e - 1, 0), 0, 0)),
            pl.BlockSpec((1, 256, 1), lambda e, t: (jnp.maximum(e - 1, 0), t, 0)),
        ],
        out_specs=pl.BlockSpec((256, H), lambda e, t: (t, 0)),
        out_shape=jax.ShapeDtypeStruct((T, H), jnp.float32),
        scratch_shapes=[pltpu.VMEM((T, H), jnp.float32)],
    )(x, w0, deltas, p)


def kernel(hidden_states, base_weight, gate_weight, mask_logits, atoms,
           expert_atom_weights, importance):
    x = hidden_states
    logits, rw, tidx, p, aux, weighted = _router(
        x, gate_weight, expert_atom_weights, atoms, importance)

    noisy = jnp.clip(mask_logits, -50.0, 50.0)
    _, midx = lax.top_k(noisy, NS)
    planes = jnp.zeros((E, NP), jnp.float32).at[:, midx].set(weighted)
    deltas = planes.reshape(E, H, H)

    pt = p.T.reshape(E, T, 1)
    out = _matmul(x, base_weight, deltas, pt)
    return (out, logits, rw, tidx, aux[0, 0])


# speedup vs baseline: 4.8439x; 4.6084x over previous
import functools

import jax
import jax.numpy as jnp
from jax import lax
from jax.experimental import pallas as pl
from jax.experimental.pallas import tpu as pltpu
from jax.experimental.pallas import tpu_sc as plsc

H = 1024
E = 8
K = 2
A = 64
NS = 16384
NP = H * H
T = 2048
NTILE = 16
PER_TILE = NP // NTILE
CAP = NS + 256
CAND = NS + 2048
DUMP = NS


def _router_body(x_ref, gw_ref, eaw_ref, atoms_ref, imp_ref,
                 logits_ref, rw_ref, tidx_ref, p_ref, aux_ref, w_ref):
    x = x_ref[...]
    gw = gw_ref[...]
    logits = lax.dot_general(x, gw, (((1,), (1,)), ((), ())),
                             preferred_element_type=jnp.float32)
    logits = jnp.clip(logits, -50.0, 50.0)
    logits_ref[...] = logits

    iota_e = lax.broadcasted_iota(jnp.int32, (T, E), 1)
    cnt = jnp.zeros((T, E), jnp.int32)
    for ep in range(E):
        col = logits[:, ep:ep + 1]
        gt = col > logits
        eq = (col == logits) & (ep < iota_e)
        cnt = cnt + (gt | eq).astype(jnp.int32)
    is0 = cnt == 0
    is1 = cnt == 1
    i0 = jnp.sum(jnp.where(is0, iota_e, 0), axis=1, keepdims=True)
    i1 = jnp.sum(jnp.where(is1, iota_e, 0), axis=1, keepdims=True)
    w0 = jnp.sum(jnp.where(is0, logits, 0.0), axis=1, keepdims=True)
    w1 = jnp.sum(jnp.where(is1, logits, 0.0), axis=1, keepdims=True)
    m = jnp.maximum(w0, w1)
    e0 = jnp.exp(w0 - m)
    e1 = jnp.exp(w1 - m)
    rw0 = e0 / (e0 + e1)
    rw1 = e1 / (e0 + e1)
    rw_ref[...] = jnp.concatenate([rw0, rw1], axis=1)
    tidx_ref[...] = jnp.concatenate([i0, i1], axis=1)
    p_ref[...] = jnp.where(is0, rw0, jnp.where(is1, rw1, 0.0))

    lmax = jnp.max(logits, axis=1, keepdims=True)
    ex = jnp.exp(logits - lmax)
    probs = ex / jnp.sum(ex, axis=1, keepdims=True)
    prob_mean = jnp.mean(probs, axis=0, keepdims=True)
    frac = jnp.mean((is0 | is1).astype(jnp.float32), axis=0, keepdims=True)
    aux_ref[...] = float(E) * jnp.sum(frac * prob_mean, axis=1, keepdims=True)

    eaw = eaw_ref[...]
    emax = jnp.max(eaw, axis=1, keepdims=True)
    eex = jnp.exp(eaw - emax)
    aw = eex / jnp.sum(eex, axis=1, keepdims=True)
    ad = lax.dot_general(aw, atoms_ref[...], (((1,), (0,)), ((), ())),
                         preferred_element_type=jnp.float32)
    w_ref[...] = imp_ref[...] * ad


def _router(x, gw, eaw, atoms, imp):
    return pl.pallas_call(
        _router_body,
        out_shape=(
            jax.ShapeDtypeStruct((T, E), jnp.float32),
            jax.ShapeDtypeStruct((T, K), jnp.float32),
            jax.ShapeDtypeStruct((T, K), jnp.int32),
            jax.ShapeDtypeStruct((T, E), jnp.float32),
            jax.ShapeDtypeStruct((1, 1), jnp.float32),
            jax.ShapeDtypeStruct((E, NS), jnp.float32),
        ),
    )(x, gw, eaw, atoms, imp)


def _thresh_body(ml_ref, keys_ref, thr_ref):
    v = jnp.clip(ml_ref[...], -50.0, 50.0)
    b = lax.bitcast_convert_type(v, jnp.int32)
    flip = jnp.where(b < 0, jnp.int32(-1), jnp.int32(-2147483648))
    keys = lax.bitcast_convert_type(b ^ flip, jnp.uint32)
    keys_ref[...] = lax.bitcast_convert_type(
        keys ^ jnp.uint32(0x80000000), jnp.int32)

    def step(i, t):
        bit = lax.shift_left(jnp.uint32(1), (jnp.uint32(31) - i.astype(jnp.uint32)))
        cand = t | bit
        n = jnp.sum((keys >= cand).astype(jnp.int32))
        return jnp.where(n >= NS, cand, t)

    thr = lax.fori_loop(0, 32, step, jnp.uint32(0))
    thr_ref[...] = jnp.full((8, 128),
                            lax.bitcast_convert_type(
                                thr ^ jnp.uint32(0x80000000), jnp.int32),
                            jnp.int32)


def _thresh(ml2d):
    return pl.pallas_call(
        _thresh_body,
        out_shape=(
            jax.ShapeDtypeStruct((8192, 128), jnp.int32),
            jax.ShapeDtypeStruct((8, 128), jnp.int32),
        ),
    )(ml2d)


def _compact_body(keys_hbm, thr_hbm, ck_hbm, ci_hbm, cnts_hbm,
                  kbuf, gk, gi, ei, thrv, rowv, all16, idxbuf, thrbuf, sem):
    w = lax.axis_index("s")
    i16 = lax.iota(jnp.int32, 16)

    pltpu.sync_copy(thr_hbm.at[pl.ds(0, 16)], thrv)
    thr = thrv[...]

    def scan_chunk(ch, carry):
        gt_n, eq_n = carry
        pltpu.sync_copy(keys_hbm.at[pl.ds(w * PER_TILE + ch * 8192, 8192)], kbuf)

        def scan_vreg(i, c):
            g, q = c
            kv = kbuf[pl.ds(i * 16, 16)]
            m_gt = kv > thr
            m_eq = kv == thr
            idxv = w * PER_TILE + ch * 8192 + i * 16 + i16
            gi_int = m_gt.astype(jnp.int32)
            qi_int = m_eq.astype(jnp.int32)
            gpos = jnp.minimum(g + plsc.cumsum(gi_int) - gi_int, CAP - 1)
            qpos = jnp.minimum(q + plsc.cumsum(qi_int) - qi_int, CAP - 1)
            plsc.store_scatter(gk, [gpos], kv, mask=m_gt)
            plsc.store_scatter(gi, [gpos], idxv, mask=m_gt)
            plsc.store_scatter(ei, [qpos], idxv, mask=m_eq)
            g = g + jnp.sum(gi_int)
            q = q + jnp.sum(qi_int)
            return g, q

        return lax.fori_loop(0, 512, scan_vreg, (gt_n, eq_n))

    gt_n, eq_n = lax.fori_loop(0, PER_TILE // 8192, scan_chunk,
                               (jnp.int32(0), jnp.int32(0)))

    rowv[...] = jnp.where(i16 == 0, gt_n, jnp.where(i16 == 1, eq_n, 0))
    pltpu.sync_copy(rowv, cnts_hbm.at[w])
    plsc.subcore_barrier()
    pltpu.sync_copy(cnts_hbm, all16)

    gt_before = jnp.int32(0)
    eq_before = jnp.int32(0)
    g_total = jnp.int32(0)
    for t in range(NTILE):
        row = all16[t]
        g_t = jnp.sum(jnp.where(i16 == 0, row, 0))
        q_t = jnp.sum(jnp.where(i16 == 1, row, 0))
        gt_before = gt_before + jnp.where(t < w, g_t, 0)
        eq_before = eq_before + jnp.where(t < w, q_t, 0)
        g_total = g_total + g_t
    k_more = NS - g_total
    take = jnp.clip(k_more - eq_before, 0, eq_n)
    eq_dest = g_total + jnp.clip(eq_before, 0, k_more)

    def fill8(r, base, n, dest0):
        for kk in range(8):
            pos = r * 128 + kk * 16 + i16
            dv = jnp.where(pos < n, dest0 + pos, DUMP + (pos & 2047))
            idxbuf[pl.ds(kk * 16, 16)] = dv

    def gt_row(r, _):
        fill8(r, 0, gt_n, gt_before)
        pltpu.async_copy(gk.at[pl.ds(r * 128, 128)], ck_hbm.at[idxbuf], sem).wait()
        pltpu.async_copy(gi.at[pl.ds(r * 128, 128)], ci_hbm.at[idxbuf], sem).wait()
        return 0

    lax.fori_loop(0, (gt_n + 127) // 128, gt_row, 0)

    def ft(kk, _):
        thrbuf[pl.ds(kk * 16, 16)] = thr
        return 0
    lax.fori_loop(0, 8, ft, 0)

    def eq_row(r, _):
        fill8(r, 0, take, eq_dest)
        pltpu.async_copy(thrbuf, ck_hbm.at[idxbuf], sem).wait()
        pltpu.async_copy(ei.at[pl.ds(r * 128, 128)], ci_hbm.at[idxbuf], sem).wait()
        return 0

    lax.fori_loop(0, (take + 127) // 128, eq_row, 0)


def _compact(keys_flat, thr_flat):
    mesh = plsc.VectorSubcoreMesh(core_axis_name="c", subcore_axis_name="s",
                                  num_cores=1, num_subcores=NTILE)
    f = pl.kernel(
        _compact_body,
        out_type=(
            jax.ShapeDtypeStruct((CAND,), jnp.int32),
            jax.ShapeDtypeStruct((CAND,), jnp.int32),
            jax.ShapeDtypeStruct((NTILE, 16), jnp.int32),
        ),
        mesh=mesh,
        scratch_types=[
            pltpu.VMEM((8192,), jnp.int32),
            pltpu.VMEM((CAP,), jnp.int32),
            pltpu.VMEM((CAP,), jnp.int32),
            pltpu.VMEM((CAP,), jnp.int32),
            pltpu.VMEM((16,), jnp.int32),
            pltpu.VMEM((16,), jnp.int32),
            pltpu.VMEM((NTILE, 16), jnp.int32),
            pltpu.VMEM((128,), jnp.int32),
            pltpu.VMEM((128,), jnp.int32),
            pltpu.SemaphoreType.DMA,
        ],
        compiler_params=pltpu.CompilerParams(needs_layout_passes=False),
    )
    return f(keys_flat, thr_flat)


def _sort_body(ck_ref, ci_ref, mi_ref):
    kk = ck_ref[...]
    vv = ci_ref[...]
    r = lax.broadcasted_iota(jnp.int32, (128, 128), 0)
    c = lax.broadcasted_iota(jnp.int32, (128, 128), 1)
    i = r * 128 + c

    for kstage in [2 ** p for p in range(1, 15)]:
        asc = (i & kstage) == 0
        j = kstage // 2
        while j >= 1:
            if j < 128:
                lowsel = (c & j) == 0
                pk = jnp.where(lowsel, pltpu.roll(kk, 128 - j, 1),
                               pltpu.roll(kk, j, 1))
                pv = jnp.where(lowsel, pltpu.roll(vv, 128 - j, 1),
                               pltpu.roll(vv, j, 1))
            else:
                jr = j // 128
                lowsel = (r & jr) == 0
                pk = jnp.where(lowsel, pltpu.roll(kk, 128 - jr, 0),
                               pltpu.roll(kk, jr, 0))
                pv = jnp.where(lowsel, pltpu.roll(vv, 128 - jr, 0),
                               pltpu.roll(vv, jr, 0))
            lt = (kk > pk) | ((kk == pk) & (vv < pv))
            keep = lt == (asc == lowsel)
            kk = jnp.where(keep, kk, pk)
            vv = jnp.where(keep, vv, pv)
            j //= 2
    mi_ref[...] = vv


def _sort(ck, ci):
    return pl.pallas_call(
        _sort_body,
        out_shape=jax.ShapeDtypeStruct((128, 128), jnp.int32),
    )(ck, ci)


def _scatter_body(midx_hbm, w_hbm, planes_hbm, zbuf, mbuf, wbuf, idxbuf, sem):
    w = lax.axis_index("s")

    def zfill(i, _):
        zbuf[pl.ds(i * 16, 16)] = jnp.zeros((16,), jnp.float32)
        return 0
    lax.fori_loop(0, 512, zfill, 0)

    base = w * (NP * E // NTILE)
    for grp in range(8):
        cps = [pltpu.async_copy(
            zbuf, planes_hbm.at[pl.ds(base + (grp * 8 + ch) * 8192, 8192)], sem)
            for ch in range(8)]
        for cp in cps:
            cp.wait()

    plsc.subcore_barrier()

    pltpu.sync_copy(midx_hbm.at[pl.ds(w * (NS // NTILE), NS // NTILE)], mbuf)
    for e in range(E):
        pltpu.sync_copy(w_hbm.at[e, pl.ds(w * (NS // NTILE), NS // NTILE)], wbuf)
        for ch in range(NS // NTILE // 128):
            def fill(kk, _):
                mv = mbuf[pl.ds(ch * 128 + kk * 16, 16)]
                idxbuf[pl.ds(kk * 16, 16)] = mv + e * NP
                return 0
            lax.fori_loop(0, 8, fill, 0)
            pltpu.async_copy(wbuf.at[pl.ds(ch * 128, 128)],
                             planes_hbm.at[idxbuf], sem).wait()


def _scatter(midx, weighted):
    mesh = plsc.VectorSubcoreMesh(core_axis_name="c", subcore_axis_name="s",
                                  num_cores=1, num_subcores=NTILE)
    f = pl.kernel(
        _scatter_body,
        out_type=jax.ShapeDtypeStruct((E * NP,), jnp.float32),
        mesh=mesh,
        scratch_types=[
            pltpu.VMEM((8192,), jnp.float32),
            pltpu.VMEM((NS // NTILE,), jnp.int32),
            pltpu.VMEM((NS // NTILE,), jnp.float32),
            pltpu.VMEM((128,), jnp.int32),
            pltpu.SemaphoreType.DMA,
        ],
        compiler_params=pltpu.CompilerParams(needs_layout_passes=False),
    )
    return f(midx, weighted)


def _matmul_body(x_ref, w0_ref, d_ref, p_ref, out_ref, acc_ref):
    e = pl.program_id(0)
    t = pl.program_id(1)
    xb = x_ref[pl.ds(t * 256, 256), :].astype(jnp.bfloat16)

    @pl.when(e == 0)
    def _():
        w = w0_ref[...].astype(jnp.bfloat16)
        acc_ref[pl.ds(t * 256, 256), :] = lax.dot_general(
            xb, w, (((1,), (1,)), ((), ())), preferred_element_type=jnp.float32)

    @pl.when(e > 0)
    def _():
        w = d_ref[0].astype(jnp.bfloat16)
        y = lax.dot_general(xb, w, (((1,), (1,)), ((), ())),
                            preferred_element_type=jnp.float32)
        coef = p_ref[0]
        acc_ref[pl.ds(t * 256, 256), :] += coef * y

    @pl.when(e == E)
    def _():
        out_ref[...] = acc_ref[pl.ds(t * 256, 256), :]


def _matmul(x, w0, deltas, p):
    grid = (E + 1, T // 256)
    return pl.pallas_call(
        _matmul_body,
        grid=grid,
        in_specs=[
            pl.BlockSpec((T, H), lambda e, t: (0, 0)),
            pl.BlockSpec((H, H), lambda e, t: (0, 0)),
            pl.BlockSpec((1, H, H), lambda e, t: (jnp.maximum(e - 1, 0), 0, 0)),
            pl.BlockSpec((1, 256, 1), lambda e, t: (jnp.maximum(e - 1, 0), t, 0)),
        ],
        out_specs=pl.BlockSpec((256, H), lambda e, t: (t, 0)),
        out_shape=jax.ShapeDtypeStruct((T, H), jnp.float32),
        scratch_shapes=[pltpu.VMEM((T, H), jnp.float32)],
    )(x, w0, deltas, p)


def kernel(hidden_states, base_weight, gate_weight, mask_logits, atoms,
           expert_atom_weights, importance):
    x = hidden_states
    logits, rw, tidx, p, aux, weighted = _router(
        x, gate_weight, expert_atom_weights, atoms, importance)

    keys2d, thr2d = _thresh(mask_logits.reshape(8192, 128))
    ck, ci, _ = _compact(keys2d.reshape(NP), thr2d.reshape(1024))
    midx2d = _sort(ck[:NS].reshape(128, 128), ci[:NS].reshape(128, 128))
    planes = _scatter(midx2d.reshape(NS), weighted)
    deltas = planes.reshape(E, H, H)

    pt = p.T.reshape(E, T, 1)
    out = _matmul(x, base_weight, deltas, pt)
    return (out, logits, rw, tidx, aux[0, 0])


# speedup vs baseline: 4.9411x; 1.0201x over previous
import functools

import jax
import jax.numpy as jnp
from jax import lax
from jax.experimental import pallas as pl
from jax.experimental.pallas import tpu as pltpu
from jax.experimental.pallas import tpu_sc as plsc

H = 1024
E = 8
K = 2
A = 64
NS = 16384
NP = H * H
T = 2048
NTILE = 16
PER_TILE = NP // NTILE
CAP = NS + 256
CAND = NS + 2048
DUMP = NS


def _router_body(x_ref, gw_ref, eaw_ref, atoms_ref, imp_ref,
                 logits_ref, rw_ref, tidx_ref, p_ref, aux_ref, w_ref):
    x = x_ref[...]
    gw = gw_ref[...]
    logits = lax.dot_general(x, gw, (((1,), (1,)), ((), ())),
                             preferred_element_type=jnp.float32)
    logits = jnp.clip(logits, -50.0, 50.0)
    logits_ref[...] = logits

    iota_e = lax.broadcasted_iota(jnp.int32, (T, E), 1)
    cnt = jnp.zeros((T, E), jnp.int32)
    for ep in range(E):
        col = logits[:, ep:ep + 1]
        gt = col > logits
        eq = (col == logits) & (ep < iota_e)
        cnt = cnt + (gt | eq).astype(jnp.int32)
    is0 = cnt == 0
    is1 = cnt == 1
    i0 = jnp.sum(jnp.where(is0, iota_e, 0), axis=1, keepdims=True)
    i1 = jnp.sum(jnp.where(is1, iota_e, 0), axis=1, keepdims=True)
    w0 = jnp.sum(jnp.where(is0, logits, 0.0), axis=1, keepdims=True)
    w1 = jnp.sum(jnp.where(is1, logits, 0.0), axis=1, keepdims=True)
    m = jnp.maximum(w0, w1)
    e0 = jnp.exp(w0 - m)
    e1 = jnp.exp(w1 - m)
    rw0 = e0 / (e0 + e1)
    rw1 = e1 / (e0 + e1)
    rw_ref[...] = jnp.concatenate([rw0, rw1], axis=1)
    tidx_ref[...] = jnp.concatenate([i0, i1], axis=1)
    p_ref[...] = jnp.where(is0, rw0, jnp.where(is1, rw1, 0.0))

    lmax = jnp.max(logits, axis=1, keepdims=True)
    ex = jnp.exp(logits - lmax)
    probs = ex / jnp.sum(ex, axis=1, keepdims=True)
    prob_mean = jnp.mean(probs, axis=0, keepdims=True)
    frac = jnp.mean((is0 | is1).astype(jnp.float32), axis=0, keepdims=True)
    aux_ref[...] = float(E) * jnp.sum(frac * prob_mean, axis=1, keepdims=True)

    eaw = eaw_ref[...]
    emax = jnp.max(eaw, axis=1, keepdims=True)
    eex = jnp.exp(eaw - emax)
    aw = eex / jnp.sum(eex, axis=1, keepdims=True)
    ad = lax.dot_general(aw, atoms_ref[...], (((1,), (0,)), ((), ())),
                         preferred_element_type=jnp.float32)
    w_ref[...] = imp_ref[...] * ad


def _router(x, gw, eaw, atoms, imp):
    return pl.pallas_call(
        _router_body,
        out_shape=(
            jax.ShapeDtypeStruct((T, E), jnp.float32),
            jax.ShapeDtypeStruct((T, K), jnp.float32),
            jax.ShapeDtypeStruct((T, K), jnp.int32),
            jax.ShapeDtypeStruct((T, E), jnp.float32),
            jax.ShapeDtypeStruct((1, 1), jnp.float32),
            jax.ShapeDtypeStruct((E, NS), jnp.float32),
        ),
    )(x, gw, eaw, atoms, imp)


def _thresh_body(ml_ref, keys_ref, thr_ref):
    v = jnp.clip(ml_ref[...], -50.0, 50.0)
    b = lax.bitcast_convert_type(v, jnp.int32)
    flip = jnp.where(b < 0, jnp.int32(-1), jnp.int32(-2147483648))
    keys = lax.bitcast_convert_type(b ^ flip, jnp.uint32)
    keys_ref[...] = lax.bitcast_convert_type(
        keys ^ jnp.uint32(0x80000000), jnp.int32)

    def step(i, t):
        bit = lax.shift_left(jnp.uint32(1), (jnp.uint32(31) - i.astype(jnp.uint32)))
        cand = t | bit
        n = jnp.sum((keys >= cand).astype(jnp.int32))
        return jnp.where(n >= NS, cand, t)

    thr = lax.fori_loop(0, 32, step, jnp.uint32(0))
    thr_ref[...] = jnp.full((8, 128),
                            lax.bitcast_convert_type(
                                thr ^ jnp.uint32(0x80000000), jnp.int32),
                            jnp.int32)


def _thresh(ml2d):
    return pl.pallas_call(
        _thresh_body,
        out_shape=(
            jax.ShapeDtypeStruct((8192, 128), jnp.int32),
            jax.ShapeDtypeStruct((8, 128), jnp.int32),
        ),
    )(ml2d)


def _lane_splat(v, lane):
    idx = jnp.full((16,), lane, jnp.int32)
    return lax.gather(v, idx.reshape(16, 1),
                      lax.GatherDimensionNumbers(
                          offset_dims=(), collapsed_slice_dims=(0,),
                          start_index_map=(0,)),
                      (1,), mode=lax.GatherScatterMode.PROMISE_IN_BOUNDS)


def _compact_body(keys_hbm, thr_hbm, ck_hbm, ci_hbm, cnts_hbm,
                  kbuf, kbuf2, gk, gi, ei, thrv, rowv, all16, idxbuf, thrbuf,
                  sem, sem2):
    w = lax.axis_index("s")
    i16 = lax.iota(jnp.int32, 16)

    pltpu.sync_copy(thr_hbm.at[pl.ds(0, 16)], thrv)
    thr = thrv[...]

    nch = PER_TILE // 8192
    bufs = [kbuf, kbuf2]
    sems = [sem, sem2]
    cur = pltpu.async_copy(keys_hbm.at[pl.ds(w * PER_TILE, 8192)], kbuf, sem)
    g_vec = jnp.zeros((16,), jnp.int32)
    q_vec = jnp.zeros((16,), jnp.int32)
    for ch in range(nch):
        nxt = None
        if ch + 1 < nch:
            nxt = pltpu.async_copy(
                keys_hbm.at[pl.ds(w * PER_TILE + (ch + 1) * 8192, 8192)],
                bufs[(ch + 1) % 2], sems[(ch + 1) % 2])
        cur.wait()
        kb = bufs[ch % 2]

        def scan_vreg(i, c):
            g, q = c
            kv = kb[pl.ds(i * 16, 16)]
            m_gt = kv > thr
            m_eq = kv == thr
            idxv = w * PER_TILE + ch * 8192 + i * 16 + i16
            gi_int = m_gt.astype(jnp.int32)
            qi_int = m_eq.astype(jnp.int32)
            csg = plsc.cumsum(gi_int)
            csq = plsc.cumsum(qi_int)
            gpos = jnp.minimum(g + csg - gi_int, CAP - 1)
            qpos = jnp.minimum(q + csq - qi_int, CAP - 1)
            plsc.store_scatter(gk, [gpos], kv, mask=m_gt)
            plsc.store_scatter(gi, [gpos], idxv, mask=m_gt)
            plsc.store_scatter(ei, [qpos], idxv, mask=m_eq)
            return g + _lane_splat(csg, 15), q + _lane_splat(csq, 15)

        g_vec, q_vec = lax.fori_loop(0, 512, scan_vreg, (g_vec, q_vec))
        cur = nxt

    rowv[...] = jnp.where(i16 == 0, g_vec, jnp.where(i16 == 1, q_vec, 0))
    pltpu.sync_copy(rowv, cnts_hbm.at[w])
    plsc.subcore_barrier()
    pltpu.sync_copy(cnts_hbm, all16)

    zero = jnp.zeros((16,), jnp.int32)
    gt_before = zero
    eq_before = zero
    g_total = zero
    for t in range(NTILE):
        row = all16[t]
        g_t = _lane_splat(row, 0)
        q_t = _lane_splat(row, 1)
        gt_before = gt_before + jnp.where(t < w, g_t, zero)
        eq_before = eq_before + jnp.where(t < w, q_t, zero)
        g_total = g_total + g_t
    k_more = NS - g_total
    take = jnp.clip(k_more - eq_before, 0, q_vec)
    eq_dest = g_total + jnp.clip(eq_before, 0, k_more)

    gt_n_s = jnp.sum(jnp.where(i16 == 0, g_vec, 0))
    take_s = jnp.sum(jnp.where(i16 == 0, take, 0))

    def fill8(r, n_vec, dest0):
        for kk in range(8):
            pos = r * 128 + kk * 16 + i16
            dv = jnp.where(pos < n_vec, dest0 + pos, DUMP + (pos & 2047))
            idxbuf[pl.ds(kk * 16, 16)] = dv

    def gt_row(r, _):
        fill8(r, g_vec, gt_before)
        c1 = pltpu.async_copy(gk.at[pl.ds(r * 128, 128)], ck_hbm.at[idxbuf], sem)
        c2 = pltpu.async_copy(gi.at[pl.ds(r * 128, 128)], ci_hbm.at[idxbuf], sem2)
        c1.wait()
        c2.wait()
        return 0

    lax.fori_loop(0, (gt_n_s + 127) // 128, gt_row, 0)

    def ft(kk, _):
        thrbuf[pl.ds(kk * 16, 16)] = thr
        return 0
    lax.fori_loop(0, 8, ft, 0)

    def eq_row(r, _):
        fill8(r, take, eq_dest)
        c1 = pltpu.async_copy(thrbuf, ck_hbm.at[idxbuf], sem)
        c2 = pltpu.async_copy(ei.at[pl.ds(r * 128, 128)], ci_hbm.at[idxbuf], sem2)
        c1.wait()
        c2.wait()
        return 0

    lax.fori_loop(0, (take_s + 127) // 128, eq_row, 0)


def _compact(keys_flat, thr_flat):
    mesh = plsc.VectorSubcoreMesh(core_axis_name="c", subcore_axis_name="s",
                                  num_cores=1, num_subcores=NTILE)
    f = pl.kernel(
        _compact_body,
        out_type=(
            jax.ShapeDtypeStruct((CAND,), jnp.int32),
            jax.ShapeDtypeStruct((CAND,), jnp.int32),
            jax.ShapeDtypeStruct((NTILE, 16), jnp.int32),
        ),
        mesh=mesh,
        scratch_types=[
            pltpu.VMEM((8192,), jnp.int32),
            pltpu.VMEM((8192,), jnp.int32),
            pltpu.VMEM((CAP,), jnp.int32),
            pltpu.VMEM((CAP,), jnp.int32),
            pltpu.VMEM((CAP,), jnp.int32),
            pltpu.VMEM((16,), jnp.int32),
            pltpu.VMEM((16,), jnp.int32),
            pltpu.VMEM((NTILE, 16), jnp.int32),
            pltpu.VMEM((128,), jnp.int32),
            pltpu.VMEM((128,), jnp.int32),
            pltpu.SemaphoreType.DMA,
            pltpu.SemaphoreType.DMA,
        ],
        compiler_params=pltpu.CompilerParams(needs_layout_passes=False),
    )
    return f(keys_flat, thr_flat)


def _sort_body(ck_ref, ci_ref, mi_ref):
    kk = ck_ref[...]
    vv = ci_ref[...]
    r = lax.broadcasted_iota(jnp.int32, (128, 128), 0)
    c = lax.broadcasted_iota(jnp.int32, (128, 128), 1)
    i = r * 128 + c

    for kstage in [2 ** p for p in range(1, 15)]:
        asc = (i & kstage) == 0
        j = kstage // 2
        while j >= 1:
            if j < 128:
                lowsel = (c & j) == 0
                pk = jnp.where(lowsel, pltpu.roll(kk, 128 - j, 1),
                               pltpu.roll(kk, j, 1))
                pv = jnp.where(lowsel, pltpu.roll(vv, 128 - j, 1),
                               pltpu.roll(vv, j, 1))
            else:
                jr = j // 128
                lowsel = (r & jr) == 0
                pk = jnp.where(lowsel, pltpu.roll(kk, 128 - jr, 0),
                               pltpu.roll(kk, jr, 0))
                pv = jnp.where(lowsel, pltpu.roll(vv, 128 - jr, 0),
                               pltpu.roll(vv, jr, 0))
            lt = (kk > pk) | ((kk == pk) & (vv < pv))
            keep = lt == (asc == lowsel)
            kk = jnp.where(keep, kk, pk)
            vv = jnp.where(keep, vv, pv)
            j //= 2
    mi_ref[...] = vv


def _sort(ck, ci):
    return pl.pallas_call(
        _sort_body,
        out_shape=jax.ShapeDtypeStruct((128, 128), jnp.int32),
    )(ck, ci)


def _scatter_body(midx_hbm, w_hbm, planes_hbm, zbuf, mbuf, wball, idxall,
                  sem, sem2):
    c = lax.axis_index("c")
    s = lax.axis_index("s")
    seg = NS // NTILE

    def zfill(i, _):
        zbuf[pl.ds(i * 16, 16)] = jnp.zeros((16,), jnp.float32)
        return 0
    lax.fori_loop(0, 512, zfill, 0)

    base = c * (4 * NP) + s * (4 * NP // NTILE)
    zcps = []
    for ch in range(32):
        zcps.append(pltpu.async_copy(
            zbuf, planes_hbm.at[pl.ds(base + ch * 8192, 8192)], sem))
        if len(zcps) > 16:
            zcps.pop(0).wait()

    pltpu.sync_copy(midx_hbm.at[pl.ds(s * seg, seg)], mbuf)
    for j in range(4):
        pltpu.sync_copy(w_hbm.at[c * 4 + j, pl.ds(s * seg, seg)], wball.at[j])
    for j in range(4):
        for ch in range(seg // 128):
            def fill(kk, _):
                mv = mbuf[pl.ds(ch * 128 + kk * 16, 16)]
                idxall[j * (seg // 128) + ch, pl.ds(kk * 16, 16)] = (
                    mv + (c * 4 + j) * NP)
                return 0
            lax.fori_loop(0, 8, fill, 0)

    for cp in zcps:
        cp.wait()
    plsc.subcore_barrier()

    scps = []
    for j in range(4):
        for ch in range(seg // 128):
            scps.append(pltpu.async_copy(
                wball.at[j, pl.ds(ch * 128, 128)],
                planes_hbm.at[idxall.at[j * (seg // 128) + ch]], sem2))
            if len(scps) > 16:
                scps.pop(0).wait()
    for cp in scps:
        cp.wait()


def _scatter(midx, weighted):
    mesh = plsc.VectorSubcoreMesh(core_axis_name="c", subcore_axis_name="s",
                                  num_cores=2, num_subcores=NTILE)
    f = pl.kernel(
        _scatter_body,
        out_type=jax.ShapeDtypeStruct((E * NP,), jnp.float32),
        mesh=mesh,
        scratch_types=[
            pltpu.VMEM((8192,), jnp.float32),
            pltpu.VMEM((NS // NTILE,), jnp.int32),
            pltpu.VMEM((4, NS // NTILE), jnp.float32),
            pltpu.VMEM((4 * (NS // NTILE // 128), 128), jnp.int32),
            pltpu.SemaphoreType.DMA,
            pltpu.SemaphoreType.DMA,
        ],
        compiler_params=pltpu.CompilerParams(needs_layout_passes=False),
    )
    return f(midx, weighted)


def _matmul_body(x_ref, w0_ref, d_ref, p_ref, out_ref, acc_ref):
    e = pl.program_id(0)
    t = pl.program_id(1)
    xb = x_ref[pl.ds(t * 256, 256), :].astype(jnp.bfloat16)

    @pl.when(e == 0)
    def _():
        w = w0_ref[...].astype(jnp.bfloat16)
        acc_ref[pl.ds(t * 256, 256), :] = lax.dot_general(
            xb, w, (((1,), (1,)), ((), ())), preferred_element_type=jnp.float32)

    @pl.when(e > 0)
    def _():
        w = d_ref[0].astype(jnp.bfloat16)
        y = lax.dot_general(xb, w, (((1,), (1,)), ((), ())),
                            preferred_element_type=jnp.float32)
        coef = p_ref[0]
        acc_ref[pl.ds(t * 256, 256), :] += coef * y

    @pl.when(e == E)
    def _():
        out_ref[...] = acc_ref[pl.ds(t * 256, 256), :]


def _matmul(x, w0, deltas, p):
    grid = (E + 1, T // 256)
    return pl.pallas_call(
        _matmul_body,
        grid=grid,
        in_specs=[
            pl.BlockSpec((T, H), lambda e, t: (0, 0)),
            pl.BlockSpec((H, H), lambda e, t: (0, 0)),
            pl.BlockSpec((1, H, H), lambda e, t: (jnp.maximum(e - 1, 0), 0, 0)),
            pl.BlockSpec((1, 256, 1), lambda e, t: (jnp.maximum(e - 1, 0), t, 0)),
        ],
        out_specs=pl.BlockSpec((256, H), lambda e, t: (t, 0)),
        out_shape=jax.ShapeDtypeStruct((T, H), jnp.float32),
        scratch_shapes=[pltpu.VMEM((T, H), jnp.float32)],
    )(x, w0, deltas, p)


def kernel(hidden_states, base_weight, gate_weight, mask_logits, atoms,
           expert_atom_weights, importance):
    x = hidden_states
    logits, rw, tidx, p, aux, weighted = _router(
        x, gate_weight, expert_atom_weights, atoms, importance)

    keys2d, thr2d = _thresh(mask_logits.reshape(8192, 128))
    ck, ci, _ = _compact(keys2d.reshape(NP), thr2d.reshape(1024))
    midx2d = _sort(ck[:NS].reshape(128, 128), ci[:NS].reshape(128, 128))
    planes = _scatter(midx2d.reshape(NS), weighted)
    deltas = planes.reshape(E, H, H)

    pt = p.T.reshape(E, T, 1)
    out = _matmul(x, base_weight, deltas, pt)
    return (out, logits, rw, tidx, aux[0, 0])


# speedup vs baseline: 5.1989x; 1.0522x over previous
import functools

import jax
import jax.numpy as jnp
from jax import lax
from jax.experimental import pallas as pl
from jax.experimental.pallas import tpu as pltpu
from jax.experimental.pallas import tpu_sc as plsc

H = 1024
E = 8
K = 2
A = 64
NS = 16384
NP = H * H
T = 2048
NTILE = 16
PER_TILE = NP // NTILE
CAP = NS + 256
CAND = NS + 2048
DUMP = NS


def _router_body(x_ref, gw_ref, eaw_ref, atoms_ref, imp_ref,
                 logits_ref, rw_ref, tidx_ref, p_ref, aux_ref, w_ref):
    x = x_ref[...]
    gw = gw_ref[...]
    logits = lax.dot_general(x, gw, (((1,), (1,)), ((), ())),
                             preferred_element_type=jnp.float32)
    logits = jnp.clip(logits, -50.0, 50.0)
    logits_ref[...] = logits

    iota_e = lax.broadcasted_iota(jnp.int32, (T, E), 1)
    cnt = jnp.zeros((T, E), jnp.int32)
    for ep in range(E):
        col = logits[:, ep:ep + 1]
        gt = col > logits
        eq = (col == logits) & (ep < iota_e)
        cnt = cnt + (gt | eq).astype(jnp.int32)
    is0 = cnt == 0
    is1 = cnt == 1
    i0 = jnp.sum(jnp.where(is0, iota_e, 0), axis=1, keepdims=True)
    i1 = jnp.sum(jnp.where(is1, iota_e, 0), axis=1, keepdims=True)
    w0 = jnp.sum(jnp.where(is0, logits, 0.0), axis=1, keepdims=True)
    w1 = jnp.sum(jnp.where(is1, logits, 0.0), axis=1, keepdims=True)
    m = jnp.maximum(w0, w1)
    e0 = jnp.exp(w0 - m)
    e1 = jnp.exp(w1 - m)
    rw0 = e0 / (e0 + e1)
    rw1 = e1 / (e0 + e1)
    rw_ref[...] = jnp.concatenate([rw0, rw1], axis=1)
    tidx_ref[...] = jnp.concatenate([i0, i1], axis=1)
    p_ref[...] = jnp.where(is0, rw0, jnp.where(is1, rw1, 0.0))

    lmax = jnp.max(logits, axis=1, keepdims=True)
    ex = jnp.exp(logits - lmax)
    probs = ex / jnp.sum(ex, axis=1, keepdims=True)
    prob_mean = jnp.mean(probs, axis=0, keepdims=True)
    frac = jnp.mean((is0 | is1).astype(jnp.float32), axis=0, keepdims=True)
    aux_ref[...] = float(E) * jnp.sum(frac * prob_mean, axis=1, keepdims=True)

    eaw = eaw_ref[...]
    emax = jnp.max(eaw, axis=1, keepdims=True)
    eex = jnp.exp(eaw - emax)
    aw = eex / jnp.sum(eex, axis=1, keepdims=True)
    ad = lax.dot_general(aw, atoms_ref[...], (((1,), (0,)), ((), ())),
                         preferred_element_type=jnp.float32)
    w_ref[...] = imp_ref[...] * ad


def _router(x, gw, eaw, atoms, imp):
    return pl.pallas_call(
        _router_body,
        out_shape=(
            jax.ShapeDtypeStruct((T, E), jnp.float32),
            jax.ShapeDtypeStruct((T, K), jnp.float32),
            jax.ShapeDtypeStruct((T, K), jnp.int32),
            jax.ShapeDtypeStruct((T, E), jnp.float32),
            jax.ShapeDtypeStruct((1, 1), jnp.float32),
            jax.ShapeDtypeStruct((E, NS), jnp.float32),
        ),
    )(x, gw, eaw, atoms, imp)


def _thresh_body(ml_ref, keys_ref, thr_ref):
    v = jnp.clip(ml_ref[...], -50.0, 50.0)
    b = lax.bitcast_convert_type(v, jnp.int32)
    flip = jnp.where(b < 0, jnp.int32(-1), jnp.int32(-2147483648))
    keys = lax.bitcast_convert_type(b ^ flip, jnp.uint32)
    keys_ref[...] = lax.bitcast_convert_type(
        keys ^ jnp.uint32(0x80000000), jnp.int32)

    def step(i, t):
        bit = lax.shift_left(jnp.uint32(1), (jnp.uint32(31) - i.astype(jnp.uint32)))
        cand = t | bit
        n = jnp.sum((keys >= cand).astype(jnp.int32))
        return jnp.where(n >= NS, cand, t)

    thr = lax.fori_loop(0, 32, step, jnp.uint32(0))
    thr_ref[...] = jnp.full((8, 128),
                            lax.bitcast_convert_type(
                                thr ^ jnp.uint32(0x80000000), jnp.int32),
                            jnp.int32)


def _thresh(ml2d):
    return pl.pallas_call(
        _thresh_body,
        out_shape=(
            jax.ShapeDtypeStruct((8192, 128), jnp.int32),
            jax.ShapeDtypeStruct((8, 128), jnp.int32),
        ),
    )(ml2d)


def _lane_splat(v, lane):
    idx = jnp.full((16,), lane, jnp.int32)
    return lax.gather(v, idx.reshape(16, 1),
                      lax.GatherDimensionNumbers(
                          offset_dims=(), collapsed_slice_dims=(0,),
                          start_index_map=(0,)),
                      (1,), mode=lax.GatherScatterMode.PROMISE_IN_BOUNDS)


def _compact_body(keys_hbm, thr_hbm, ck_hbm, ci_hbm, cnts_hbm,
                  kbuf, kbuf2, gk, gi, ei, thrv, rowv, all16, idxbuf, thrbuf,
                  sem, sem2):
    w = lax.axis_index("s")
    i16 = lax.iota(jnp.int32, 16)

    pltpu.sync_copy(thr_hbm.at[pl.ds(0, 16)], thrv)
    thr = thrv[...]

    nch = PER_TILE // 8192
    bufs = [kbuf, kbuf2]
    sems = [sem, sem2]
    cur = pltpu.async_copy(keys_hbm.at[pl.ds(w * PER_TILE, 8192)], kbuf, sem)
    g_vec = jnp.zeros((16,), jnp.int32)
    q_vec = jnp.zeros((16,), jnp.int32)
    for ch in range(nch):
        nxt = None
        if ch + 1 < nch:
            nxt = pltpu.async_copy(
                keys_hbm.at[pl.ds(w * PER_TILE + (ch + 1) * 8192, 8192)],
                bufs[(ch + 1) % 2], sems[(ch + 1) % 2])
        cur.wait()
        kb = bufs[ch % 2]

        def scan_block(i, c):
            g, q = c
            kvs, bgs, bqs, mgs, mqs, csgs, csqs = [], [], [], [], [], [], []
            for u in range(4):
                kv = kb[pl.ds(i * 64 + u * 16, 16)]
                m_gt = kv > thr
                m_eq = kv == thr
                kvs.append(kv)
                bgs.append(m_gt)
                bqs.append(m_eq)
                mgs.append(m_gt.astype(jnp.int32))
                mqs.append(m_eq.astype(jnp.int32))
                csgs.append(plsc.cumsum(mgs[u]))
                csqs.append(plsc.cumsum(mqs[u]))
            gb, qb = g, q
            for u in range(4):
                idxv = w * PER_TILE + ch * 8192 + i * 64 + u * 16 + i16
                gpos = jnp.minimum(gb + csgs[u] - mgs[u], CAP - 1)
                qpos = jnp.minimum(qb + csqs[u] - mqs[u], CAP - 1)
                plsc.store_scatter(gk, [gpos], kvs[u], mask=bgs[u])
                plsc.store_scatter(gi, [gpos], idxv, mask=bgs[u])
                plsc.store_scatter(ei, [qpos], idxv, mask=bqs[u])
                gb = gb + _lane_splat(csgs[u], 15)
                qb = qb + _lane_splat(csqs[u], 15)
            return gb, qb

        g_vec, q_vec = lax.fori_loop(0, 128, scan_block, (g_vec, q_vec))
        cur = nxt

    rowv[...] = jnp.where(i16 == 0, g_vec, jnp.where(i16 == 1, q_vec, 0))
    pltpu.sync_copy(rowv, cnts_hbm.at[w])
    plsc.subcore_barrier()
    pltpu.sync_copy(cnts_hbm, all16)

    zero = jnp.zeros((16,), jnp.int32)
    gt_before = zero
    eq_before = zero
    g_total = zero
    for t in range(NTILE):
        row = all16[t]
        g_t = _lane_splat(row, 0)
        q_t = _lane_splat(row, 1)
        gt_before = gt_before + jnp.where(t < w, g_t, zero)
        eq_before = eq_before + jnp.where(t < w, q_t, zero)
        g_total = g_total + g_t
    k_more = NS - g_total
    take = jnp.clip(k_more - eq_before, 0, q_vec)
    eq_dest = g_total + jnp.clip(eq_before, 0, k_more)

    gt_n_s = jnp.sum(jnp.where(i16 == 0, g_vec, 0))
    take_s = jnp.sum(jnp.where(i16 == 0, take, 0))

    def fill8(r, n_vec, dest0):
        for kk in range(8):
            pos = r * 128 + kk * 16 + i16
            dv = jnp.where(pos < n_vec, dest0 + pos, DUMP + (pos & 2047))
            idxbuf[pl.ds(kk * 16, 16)] = dv

    def gt_row(r, _):
        fill8(r, g_vec, gt_before)
        c1 = pltpu.async_copy(gk.at[pl.ds(r * 128, 128)], ck_hbm.at[idxbuf], sem)
        c2 = pltpu.async_copy(gi.at[pl.ds(r * 128, 128)], ci_hbm.at[idxbuf], sem2)
        c1.wait()
        c2.wait()
        return 0

    lax.fori_loop(0, (gt_n_s + 127) // 128, gt_row, 0)

    def ft(kk, _):
        thrbuf[pl.ds(kk * 16, 16)] = thr
        return 0
    lax.fori_loop(0, 8, ft, 0)

    def eq_row(r, _):
        fill8(r, take, eq_dest)
        c1 = pltpu.async_copy(thrbuf, ck_hbm.at[idxbuf], sem)
        c2 = pltpu.async_copy(ei.at[pl.ds(r * 128, 128)], ci_hbm.at[idxbuf], sem2)
        c1.wait()
        c2.wait()
        return 0

    lax.fori_loop(0, (take_s + 127) // 128, eq_row, 0)


def _compact(keys_flat, thr_flat):
    mesh = plsc.VectorSubcoreMesh(core_axis_name="c", subcore_axis_name="s",
                                  num_cores=1, num_subcores=NTILE)
    f = pl.kernel(
        _compact_body,
        out_type=(
            jax.ShapeDtypeStruct((CAND,), jnp.int32),
            jax.ShapeDtypeStruct((CAND,), jnp.int32),
            jax.ShapeDtypeStruct((NTILE, 16), jnp.int32),
        ),
        mesh=mesh,
        scratch_types=[
            pltpu.VMEM((8192,), jnp.int32),
            pltpu.VMEM((8192,), jnp.int32),
            pltpu.VMEM((CAP,), jnp.int32),
            pltpu.VMEM((CAP,), jnp.int32),
            pltpu.VMEM((CAP,), jnp.int32),
            pltpu.VMEM((16,), jnp.int32),
            pltpu.VMEM((16,), jnp.int32),
            pltpu.VMEM((NTILE, 16), jnp.int32),
            pltpu.VMEM((128,), jnp.int32),
            pltpu.VMEM((128,), jnp.int32),
            pltpu.SemaphoreType.DMA,
            pltpu.SemaphoreType.DMA,
        ],
        compiler_params=pltpu.CompilerParams(needs_layout_passes=False),
    )
    return f(keys_flat, thr_flat)


def _sort_body(ck_ref, ci_ref, mi_ref):
    kk = ck_ref[...]
    vv = ci_ref[...]
    r = lax.broadcasted_iota(jnp.int32, (128, 128), 0)
    c = lax.broadcasted_iota(jnp.int32, (128, 128), 1)
    i = r * 128 + c

    for kstage in [2 ** p for p in range(1, 15)]:
        asc = (i & kstage) == 0
        j = kstage // 2
        while j >= 1:
            if j < 128:
                lowsel = (c & j) == 0
                pk = jnp.where(lowsel, pltpu.roll(kk, 128 - j, 1),
                               pltpu.roll(kk, j, 1))
                pv = jnp.where(lowsel, pltpu.roll(vv, 128 - j, 1),
                               pltpu.roll(vv, j, 1))
            else:
                jr = j // 128
                lowsel = (r & jr) == 0
                pk = jnp.where(lowsel, pltpu.roll(kk, 128 - jr, 0),
                               pltpu.roll(kk, jr, 0))
                pv = jnp.where(lowsel, pltpu.roll(vv, 128 - jr, 0),
                               pltpu.roll(vv, jr, 0))
            lt = (kk > pk) | ((kk == pk) & (vv < pv))
            keep = lt == (asc == lowsel)
            kk = jnp.where(keep, kk, pk)
            vv = jnp.where(keep, vv, pv)
            j //= 2
    mi_ref[...] = vv


def _sort(ck, ci):
    return pl.pallas_call(
        _sort_body,
        out_shape=jax.ShapeDtypeStruct((128, 128), jnp.int32),
    )(ck, ci)


def _scatter_body(midx_hbm, w_hbm, planes_hbm, zbuf, mbuf, wball, idxall,
                  sem, sem2):
    c = lax.axis_index("c")
    s = lax.axis_index("s")
    seg = NS // NTILE

    def zfill(i, _):
        zbuf[pl.ds(i * 16, 16)] = jnp.zeros((16,), jnp.float32)
        return 0
    lax.fori_loop(0, 512, zfill, 0)

    base = c * (4 * NP) + s * (4 * NP // NTILE)
    zcps = []
    for ch in range(32):
        zcps.append(pltpu.async_copy(
            zbuf, planes_hbm.at[pl.ds(base + ch * 8192, 8192)], sem))
        if len(zcps) > 16:
            zcps.pop(0).wait()

    pltpu.sync_copy(midx_hbm.at[pl.ds(s * seg, seg)], mbuf)
    for j in range(4):
        pltpu.sync_copy(w_hbm.at[c * 4 + j, pl.ds(s * seg, seg)], wball.at[j])
    for j in range(4):
        for ch in range(seg // 128):
            def fill(kk, _):
                mv = mbuf[pl.ds(ch * 128 + kk * 16, 16)]
                idxall[j * (seg // 128) + ch, pl.ds(kk * 16, 16)] = (
                    mv + (c * 4 + j) * NP)
                return 0
            lax.fori_loop(0, 8, fill, 0)

    for cp in zcps:
        cp.wait()
    plsc.subcore_barrier()

    scps = []
    for j in range(4):
        for ch in range(seg // 128):
            scps.append(pltpu.async_copy(
                wball.at[j, pl.ds(ch * 128, 128)],
                planes_hbm.at[idxall.at[j * (seg // 128) + ch]], sem2))
            if len(scps) > 16:
                scps.pop(0).wait()
    for cp in scps:
        cp.wait()


def _scatter(midx, weighted):
    mesh = plsc.VectorSubcoreMesh(core_axis_name="c", subcore_axis_name="s",
                                  num_cores=2, num_subcores=NTILE)
    f = pl.kernel(
        _scatter_body,
        out_type=jax.ShapeDtypeStruct((E * NP,), jnp.float32),
        mesh=mesh,
        scratch_types=[
            pltpu.VMEM((8192,), jnp.float32),
            pltpu.VMEM((NS // NTILE,), jnp.int32),
            pltpu.VMEM((4, NS // NTILE), jnp.float32),
            pltpu.VMEM((4 * (NS // NTILE // 128), 128), jnp.int32),
            pltpu.SemaphoreType.DMA,
            pltpu.SemaphoreType.DMA,
        ],
        compiler_params=pltpu.CompilerParams(needs_layout_passes=False),
    )
    return f(midx, weighted)


def _matmul_body(x_ref, w0_ref, d_ref, p_ref, out_ref, acc_ref):
    e = pl.program_id(0)
    t = pl.program_id(1)
    xb = x_ref[pl.ds(t * 256, 256), :].astype(jnp.bfloat16)

    @pl.when(e == 0)
    def _():
        w = w0_ref[...].astype(jnp.bfloat16)
        acc_ref[pl.ds(t * 256, 256), :] = lax.dot_general(
            xb, w, (((1,), (1,)), ((), ())), preferred_element_type=jnp.float32)

    @pl.when(e > 0)
    def _():
        w = d_ref[0].astype(jnp.bfloat16)
        y = lax.dot_general(xb, w, (((1,), (1,)), ((), ())),
                            preferred_element_type=jnp.float32)
        coef = p_ref[0]
        acc_ref[pl.ds(t * 256, 256), :] += coef * y

    @pl.when(e == E)
    def _():
        out_ref[...] = acc_ref[pl.ds(t * 256, 256), :]


def _matmul(x, w0, deltas, p):
    grid = (E + 1, T // 256)
    return pl.pallas_call(
        _matmul_body,
        grid=grid,
        in_specs=[
            pl.BlockSpec((T, H), lambda e, t: (0, 0)),
            pl.BlockSpec((H, H), lambda e, t: (0, 0)),
            pl.BlockSpec((1, H, H), lambda e, t: (jnp.maximum(e - 1, 0), 0, 0)),
            pl.BlockSpec((1, 256, 1), lambda e, t: (jnp.maximum(e - 1, 0), t, 0)),
        ],
        out_specs=pl.BlockSpec((256, H), lambda e, t: (t, 0)),
        out_shape=jax.ShapeDtypeStruct((T, H), jnp.float32),
        scratch_shapes=[pltpu.VMEM((T, H), jnp.float32)],
    )(x, w0, deltas, p)


def kernel(hidden_states, base_weight, gate_weight, mask_logits, atoms,
           expert_atom_weights, importance):
    x = hidden_states
    logits, rw, tidx, p, aux, weighted = _router(
        x, gate_weight, expert_atom_weights, atoms, importance)

    keys2d, thr2d = _thresh(mask_logits.reshape(8192, 128))
    ck, ci, _ = _compact(keys2d.reshape(NP), thr2d.reshape(1024))
    midx2d = _sort(ck[:NS].reshape(128, 128), ci[:NS].reshape(128, 128))
    planes = _scatter(midx2d.reshape(NS), weighted)
    deltas = planes.reshape(E, H, H)

    pt = p.T.reshape(E, T, 1)
    out = _matmul(x, base_weight, deltas, pt)
    return (out, logits, rw, tidx, aux[0, 0])
